# Initial kernel scaffold; baseline (speedup 1.0000x reference)
import jax
import jax.numpy as jnp
from jax.experimental import pallas as pl


def kernel(h, g, batch_ids, edge_index, W1, b1, ln_g, ln_b, W2, b2):
    raise NotImplementedError("write your pallas kernel here")


# speedup vs baseline: 131.6315x; 131.6315x over previous
import functools

import jax
import jax.numpy as jnp
from jax import lax
from jax.experimental import pallas as pl
from jax.experimental.pallas import tpu as pltpu
from jax.experimental.pallas import tpu_sc as plsc

NEG = -1e30
BIG = 2**30


def _head_body(h_ref, bi_ref, g_ref, w1_ref, b1_ref, lng_ref, lnb_ref,
               w2_ref, b2_ref, logits_ref, prob_ref, seed_ref):
    nblk = pl.num_programs(0)
    i = pl.program_id(0)
    R = h_ref.shape[0]
    GB = g_ref.shape[0]

    bi = bi_ref[pl.ds(i, 1)].reshape(1, R)
    one = (bi == lax.broadcasted_iota(jnp.int32, (GB, R), 0))
    onef = one.astype(jnp.float32)
    g_rep = lax.dot_general(onef, g_ref[...], (((0,), (0,)), ((), ())),
                            preferred_element_type=jnp.float32)
    z = h_ref[...] + g_rep
    u = jnp.dot(z, w1_ref[...], preferred_element_type=jnp.float32) + b1_ref[...]
    mu = jnp.mean(u, axis=-1, keepdims=True)
    var = jnp.mean((u - mu) * (u - mu), axis=-1, keepdims=True)
    un = (u - mu) / jnp.sqrt(var + 1e-5) * lng_ref[...] + lnb_ref[...]
    ur = jnp.maximum(un, 0.0)
    lrow = lax.dot_general(w2_ref[...], ur, (((1,), (1,)), ((), ())),
                           preferred_element_type=jnp.float32) + b2_ref[...]
    logits_ref[pl.ds(i, 1)] = lrow.reshape(1, 1, R)

    @pl.when(i == nblk - 1)
    def _segment_stage():
        def chunk(c):
            l = logits_ref[pl.ds(c, 1)].reshape(1, R)
            b = bi_ref[pl.ds(c, 1)].reshape(1, R)
            o = (b == lax.broadcasted_iota(jnp.int32, (GB, R), 0))
            return l, o, o.astype(jnp.float32)

        def p1(c, mx):
            l, o, _ = chunk(c)
            return jnp.maximum(mx, jnp.max(jnp.where(o, l, NEG), axis=1,
                                           keepdims=True))
        mx = lax.fori_loop(0, nblk, p1, jnp.full((GB, 1), NEG, jnp.float32))

        def p2(c, ss):
            l, o, of = chunk(c)
            mxr = lax.dot_general(mx, of, (((0,), (0,)), ((), ())),
                                  preferred_element_type=jnp.float32)
            ex = jnp.exp(l - mxr)
            return ss + jnp.sum(jnp.where(o, ex, 0.0), axis=1, keepdims=True)
        ss = lax.fori_loop(0, nblk, p2, jnp.zeros((GB, 1), jnp.float32))

        def p3(c, pm):
            l, o, of = chunk(c)
            mxr = lax.dot_general(mx, of, (((0,), (0,)), ((), ())),
                                  preferred_element_type=jnp.float32)
            ssr = lax.dot_general(ss, of, (((0,), (0,)), ((), ())),
                                  preferred_element_type=jnp.float32)
            p = jnp.exp(l - mxr) / ssr
            prob_ref[pl.ds(c, 1)] = p.reshape(1, 1, R)
            return jnp.maximum(pm, jnp.max(jnp.where(o, p, NEG), axis=1,
                                           keepdims=True))
        pm = lax.fori_loop(0, nblk, p3, jnp.full((GB, 1), NEG, jnp.float32))

        def p4(c, im):
            _, o, _ = chunk(c)
            p = prob_ref[pl.ds(c, 1)].reshape(1, R)
            gidx = c * R + lax.broadcasted_iota(jnp.int32, (1, R), 1)
            cand = jnp.where(o & (p == pm), gidx, BIG)
            return jnp.minimum(im, jnp.min(cand, axis=1, keepdims=True))
        im = lax.fori_loop(0, nblk, p4, jnp.full((GB, 1), BIG, jnp.int32))

        def p5(c, _):
            _, o, _ = chunk(c)
            gidx = c * R + lax.broadcasted_iota(jnp.int32, (1, R), 1)
            hit = jnp.where(o & (gidx == im), 1.0, 0.0)
            seed_ref[pl.ds(c, 1)] = jnp.max(hit, axis=0, keepdims=True
                                            ).reshape(1, 1, R)
            return 0
        lax.fori_loop(0, nblk, p5, 0)


def _head_call(h, g, batch_ids, W1, b1, ln_g, ln_b, W2, b2):
    N, HID = h.shape
    B = g.shape[0]
    H2 = W1.shape[1]
    R = 1000
    nblk = N // R
    GB = 128
    g_pad = jnp.zeros((GB, HID), jnp.float32).at[:B].set(g)
    bi3 = batch_ids.astype(jnp.int32).reshape(nblk, 1, R)

    full3 = lambda s: pl.BlockSpec(s, lambda i: (0, 0, 0))
    full2 = lambda s: pl.BlockSpec(s, lambda i: (0, 0))
    outs = pl.pallas_call(
        _head_body,
        grid=(nblk,),
        in_specs=[
            pl.BlockSpec((R, HID), lambda i: (i, 0)),
            full3((nblk, 1, R)),
            full2((GB, HID)),
            full2((HID, H2)),
            full2((1, H2)),
            full2((1, H2)),
            full2((1, H2)),
            full2((1, H2)),
            full2((1, 1)),
        ],
        out_specs=[full3((nblk, 1, R)), full3((nblk, 1, R)),
                   full3((nblk, 1, R))],
        out_shape=[jax.ShapeDtypeStruct((nblk, 1, R), jnp.float32)] * 3,
    )(h, bi3, g_pad, W1, b1.reshape(1, H2), ln_g.reshape(1, H2),
      ln_b.reshape(1, H2), W2.reshape(1, H2), b2.reshape(1, 1))
    logits3, prob3, seed3 = outs
    return logits3.reshape(N), prob3.reshape(N), seed3.reshape(N)


_NPAD = 10240
_SLC = _NPAD // 16


def _edge_body(EP, src_hbm, dst_hbm, seed_hbm, logits_hbm,
               ew_hbm, em_hbm, ni_hbm,
               src_v, dst_v, out_v, fr_v, nx_v, rc_v, lg_v, ni_v,
               sh_all, sh_comb):
    sid = lax.axis_index("s")
    cid = lax.axis_index("c")
    half = EP // 2
    zeros16 = jnp.zeros((16,), jnp.float32)
    ones16 = jnp.ones((16,), jnp.float32)

    pltpu.sync_copy(src_hbm.at[pl.ds(sid * EP, EP)], src_v)
    pltpu.sync_copy(dst_hbm.at[pl.ds(sid * EP, EP)], dst_v)
    pltpu.sync_copy(seed_hbm, fr_v)
    pltpu.sync_copy(seed_hbm, rc_v)
    pltpu.sync_copy(logits_hbm, lg_v)

    @plsc.parallel_loop(0, _NPAD, 16, unroll=8)
    def _zero(t):
        t = pl.multiple_of(t, 16)
        nx_v[pl.ds(t, 16)] = zeros16
        ni_v[pl.ds(t, 16)] = zeros16

    def combine(part_v):
        pltpu.sync_copy(part_v, sh_all.at[pl.ds(sid * _NPAD, _NPAD)])
        plsc.subcore_barrier()
        for c in range(16):
            pltpu.sync_copy(sh_all.at[pl.ds(c * _NPAD + sid * _SLC, _SLC)],
                            out_v.at[pl.ds(c * _SLC, _SLC)])

        @plsc.parallel_loop(0, _SLC, 16, unroll=4)
        def _sum(t):
            t = pl.multiple_of(t, 16)
            acc = out_v[pl.ds(t, 16)]
            for c in range(1, 16):
                acc = acc + out_v[pl.ds(c * _SLC + t, 16)]
            out_v[pl.ds(t, 16)] = acc

    def hop():
        @plsc.parallel_loop(0, EP, 16, unroll=4)
        def _scan(e):
            e = pl.multiple_of(e, 16)
            d = dst_v[pl.ds(e, 16)]
            fm = plsc.load_gather(fr_v, [d])
            s = src_v[pl.ds(e, 16)]
            plsc.store_scatter(nx_v, [s], ones16, mask=fm > 0.0)

        combine(nx_v)
        pltpu.sync_copy(out_v.at[pl.ds(0, _SLC)],
                        sh_comb.at[pl.ds(sid * _SLC, _SLC)])
        plsc.subcore_barrier()
        pltpu.sync_copy(sh_comb, fr_v)

        @plsc.parallel_loop(0, _NPAD, 16, unroll=8)
        def _upd(t):
            t = pl.multiple_of(t, 16)
            rc_v[pl.ds(t, 16)] = rc_v[pl.ds(t, 16)] + fr_v[pl.ds(t, 16)]
            nx_v[pl.ds(t, 16)] = zeros16

    hop()
    hop()

    for hh in range(2):
        @plsc.parallel_loop(0, half, 16, unroll=4)
        def _fin(e):
            e = pl.multiple_of(e, 16)
            s = src_v[pl.ds(hh * half + e, 16)]
            d = dst_v[pl.ds(hh * half + e, 16)]
            rs = plsc.load_gather(rc_v, [s])
            rd = plsc.load_gather(rc_v, [d])
            m = (rs > 0.0) & (rd > 0.0)
            ls = plsc.load_gather(lg_v, [s])
            ld = plsc.load_gather(lg_v, [d])
            out_v[pl.ds(e, 16)] = jnp.where(m, ls + ld, 0.0)
            out_v[pl.ds(half + e, 16)] = jnp.where(m, ones16, zeros16)
            plsc.store_scatter(ni_v, [s], ones16, mask=m)
            plsc.store_scatter(ni_v, [d], ones16, mask=m)

        @pl.when(cid == 0)
        def _wr():
            pltpu.sync_copy(out_v.at[pl.ds(0, half)],
                            ew_hbm.at[pl.ds(sid * EP + hh * half, half)])
            pltpu.sync_copy(out_v.at[pl.ds(half, half)],
                            em_hbm.at[pl.ds(sid * EP + hh * half, half)])

    combine(ni_v)

    @pl.when(cid == 0)
    def _wr_ni():
        pltpu.sync_copy(out_v.at[pl.ds(0, _SLC)],
                        ni_hbm.at[pl.ds(sid * _SLC, _SLC)])


def _edge_call(src, dst, seed_pad, logits_pad):
    E = src.shape[0]
    EP = E // 16
    mesh = plsc.VectorSubcoreMesh(core_axis_name="c", subcore_axis_name="s",
                                  num_cores=2, num_subcores=16)
    f32 = jnp.float32
    kern = functools.partial(
        pl.kernel,
        out_type=[jax.ShapeDtypeStruct((E,), f32),
                  jax.ShapeDtypeStruct((E,), f32),
                  jax.ShapeDtypeStruct((_NPAD,), f32)],
        mesh=mesh,
        compiler_params=pltpu.CompilerParams(needs_layout_passes=False),
        scratch_types=[
            pltpu.VMEM((EP,), jnp.int32),
            pltpu.VMEM((EP,), jnp.int32),
            pltpu.VMEM((EP,), f32),
            pltpu.VMEM((_NPAD,), f32),
            pltpu.VMEM((_NPAD,), f32),
            pltpu.VMEM((_NPAD,), f32),
            pltpu.VMEM((_NPAD,), f32),
            pltpu.VMEM((_NPAD,), f32),
            pltpu.VMEM_SHARED((16 * _NPAD,), f32),
            pltpu.VMEM_SHARED((_NPAD,), f32),
        ],
    )(functools.partial(_edge_body, EP))
    return kern(src, dst, seed_pad, logits_pad)


def kernel(h, g, batch_ids, edge_index, W1, b1, ln_g, ln_b, W2, b2):
    N = h.shape[0]
    logits, node_prob, seed = _head_call(h, g, batch_ids, W1, b1,
                                         ln_g, ln_b, W2, b2)
    src = edge_index[0].astype(jnp.int32)
    dst = edge_index[1].astype(jnp.int32)
    seed_pad = jnp.zeros((_NPAD,), jnp.float32).at[:N].set(seed)
    logits_pad = jnp.zeros((_NPAD,), jnp.float32).at[:N].set(logits)
    ew, emf, nic = _edge_call(src, dst, seed_pad, logits_pad)
    edge_mask = emf > 0.0
    node_in_mask = nic[:N] > 0.0
    return ew, node_prob, edge_mask, node_in_mask


# speedup vs baseline: 134.5893x; 1.0225x over previous
import functools

import jax
import jax.numpy as jnp
from jax import lax
from jax.experimental import pallas as pl
from jax.experimental.pallas import tpu as pltpu
from jax.experimental.pallas import tpu_sc as plsc

NEG = -1e30
BIG = 2**30


def _head_body(h_ref, bi_ref, g_ref, w1_ref, b1_ref, lng_ref, lnb_ref,
               w2_ref, b2_ref, logits_ref, prob_ref, seed_ref):
    nblk = pl.num_programs(0)
    i = pl.program_id(0)
    R = h_ref.shape[0]
    GB = g_ref.shape[0]

    bi = bi_ref[pl.ds(i, 1)].reshape(1, R)
    one = (bi == lax.broadcasted_iota(jnp.int32, (GB, R), 0))
    onef = one.astype(jnp.float32)
    g_rep = lax.dot_general(onef, g_ref[...], (((0,), (0,)), ((), ())),
                            preferred_element_type=jnp.float32)
    z = h_ref[...] + g_rep
    u = jnp.dot(z, w1_ref[...], preferred_element_type=jnp.float32) + b1_ref[...]
    mu = jnp.mean(u, axis=-1, keepdims=True)
    var = jnp.mean((u - mu) * (u - mu), axis=-1, keepdims=True)
    un = (u - mu) / jnp.sqrt(var + 1e-5) * lng_ref[...] + lnb_ref[...]
    ur = jnp.maximum(un, 0.0)
    lrow = lax.dot_general(w2_ref[...], ur, (((1,), (1,)), ((), ())),
                           preferred_element_type=jnp.float32) + b2_ref[...]
    logits_ref[pl.ds(i, 1)] = lrow.reshape(1, 1, R)

    @pl.when(i == nblk - 1)
    def _segment_stage():
        def chunk(c):
            l = logits_ref[pl.ds(c, 1)].reshape(1, R)
            b = bi_ref[pl.ds(c, 1)].reshape(1, R)
            o = (b == lax.broadcasted_iota(jnp.int32, (GB, R), 0))
            return l, o, o.astype(jnp.float32)

        def p1(c, mx):
            l, o, _ = chunk(c)
            return jnp.maximum(mx, jnp.max(jnp.where(o, l, NEG), axis=1,
                                           keepdims=True))
        mx = lax.fori_loop(0, nblk, p1, jnp.full((GB, 1), NEG, jnp.float32))

        def p2(c, ss):
            l, o, of = chunk(c)
            mxr = lax.dot_general(mx, of, (((0,), (0,)), ((), ())),
                                  preferred_element_type=jnp.float32)
            ex = jnp.exp(l - mxr)
            return ss + jnp.sum(jnp.where(o, ex, 0.0), axis=1, keepdims=True)
        ss = lax.fori_loop(0, nblk, p2, jnp.zeros((GB, 1), jnp.float32))

        def p3(c, pm):
            l, o, of = chunk(c)
            mxr = lax.dot_general(mx, of, (((0,), (0,)), ((), ())),
                                  preferred_element_type=jnp.float32)
            ssr = lax.dot_general(ss, of, (((0,), (0,)), ((), ())),
                                  preferred_element_type=jnp.float32)
            p = jnp.exp(l - mxr) / ssr
            prob_ref[pl.ds(c, 1)] = p.reshape(1, 1, R)
            return jnp.maximum(pm, jnp.max(jnp.where(o, p, NEG), axis=1,
                                           keepdims=True))
        pm = lax.fori_loop(0, nblk, p3, jnp.full((GB, 1), NEG, jnp.float32))

        def p4(c, im):
            _, o, _ = chunk(c)
            p = prob_ref[pl.ds(c, 1)].reshape(1, R)
            gidx = c * R + lax.broadcasted_iota(jnp.int32, (1, R), 1)
            cand = jnp.where(o & (p == pm), gidx, BIG)
            return jnp.minimum(im, jnp.min(cand, axis=1, keepdims=True))
        im = lax.fori_loop(0, nblk, p4, jnp.full((GB, 1), BIG, jnp.int32))

        def p5(c, _):
            _, o, _ = chunk(c)
            gidx = c * R + lax.broadcasted_iota(jnp.int32, (1, R), 1)
            hit = jnp.where(o & (gidx == im), 1.0, 0.0)
            seed_ref[pl.ds(c, 1)] = jnp.max(hit, axis=0, keepdims=True
                                            ).reshape(1, 1, R)
            return 0
        lax.fori_loop(0, nblk, p5, 0)


def _head_call(h, g, batch_ids, W1, b1, ln_g, ln_b, W2, b2):
    N, HID = h.shape
    B = g.shape[0]
    H2 = W1.shape[1]
    R = 1000
    nblk = N // R
    GB = 128
    g_pad = jnp.zeros((GB, HID), jnp.float32).at[:B].set(g)
    bi3 = batch_ids.astype(jnp.int32).reshape(nblk, 1, R)

    full3 = lambda s: pl.BlockSpec(s, lambda i: (0, 0, 0))
    full2 = lambda s: pl.BlockSpec(s, lambda i: (0, 0))
    outs = pl.pallas_call(
        _head_body,
        grid=(nblk,),
        in_specs=[
            pl.BlockSpec((R, HID), lambda i: (i, 0)),
            full3((nblk, 1, R)),
            full2((GB, HID)),
            full2((HID, H2)),
            full2((1, H2)),
            full2((1, H2)),
            full2((1, H2)),
            full2((1, H2)),
            full2((1, 1)),
        ],
        out_specs=[full3((nblk, 1, R)), full3((nblk, 1, R)),
                   full3((nblk, 1, R))],
        out_shape=[jax.ShapeDtypeStruct((nblk, 1, R), jnp.float32)] * 3,
    )(h, bi3, g_pad, W1, b1.reshape(1, H2), ln_g.reshape(1, H2),
      ln_b.reshape(1, H2), W2.reshape(1, H2), b2.reshape(1, 1))
    logits3, prob3, seed3 = outs
    return logits3.reshape(N), prob3.reshape(N), seed3.reshape(N)


_NPAD = 10240
_SLC = _NPAD // 16


def _edge_body(EP, src_hbm, dst_hbm, seed_hbm, logits_hbm,
               ew_hbm, em_hbm, ni_hbm,
               src_v, dst_v, out_v, fr_v, nx_v, rc_v, lg_v, ni_v,
               sh_all, sh_comb):
    sid = lax.axis_index("s")
    cid = lax.axis_index("c")
    half = EP // 2
    zeros16 = jnp.zeros((16,), jnp.float32)
    ones16 = jnp.ones((16,), jnp.float32)

    pltpu.sync_copy(src_hbm.at[pl.ds(sid * EP, EP)], src_v)
    pltpu.sync_copy(dst_hbm.at[pl.ds(sid * EP, EP)], dst_v)
    pltpu.sync_copy(seed_hbm, fr_v)
    pltpu.sync_copy(seed_hbm, rc_v)
    pltpu.sync_copy(logits_hbm, lg_v)

    @plsc.parallel_loop(0, _NPAD, 16, unroll=8)
    def _zero(t):
        t = pl.multiple_of(t, 16)
        nx_v[pl.ds(t, 16)] = zeros16
        ni_v[pl.ds(t, 16)] = zeros16

    def combine(part_v):
        pltpu.sync_copy(part_v, sh_all.at[pl.ds(sid * _NPAD, _NPAD)])
        plsc.subcore_barrier()
        for c in range(16):
            pltpu.sync_copy(sh_all.at[pl.ds(c * _NPAD + sid * _SLC, _SLC)],
                            out_v.at[pl.ds(c * _SLC, _SLC)])

        @plsc.parallel_loop(0, _SLC, 16, unroll=4)
        def _sum(t):
            t = pl.multiple_of(t, 16)
            acc = out_v[pl.ds(t, 16)]
            for c in range(1, 16):
                acc = acc + out_v[pl.ds(c * _SLC + t, 16)]
            out_v[pl.ds(t, 16)] = acc

    def hop():
        @plsc.parallel_loop(0, EP, 16, unroll=8)
        def _scan(e):
            e = pl.multiple_of(e, 16)
            d = dst_v[pl.ds(e, 16)]
            fm = plsc.load_gather(fr_v, [d])
            s = src_v[pl.ds(e, 16)]
            plsc.store_scatter(nx_v, [s], ones16, mask=fm > 0.0)

        combine(nx_v)
        pltpu.sync_copy(out_v.at[pl.ds(0, _SLC)],
                        sh_comb.at[pl.ds(sid * _SLC, _SLC)])
        plsc.subcore_barrier()
        pltpu.sync_copy(sh_comb, fr_v)

        @plsc.parallel_loop(0, _NPAD, 16, unroll=8)
        def _upd(t):
            t = pl.multiple_of(t, 16)
            rc_v[pl.ds(t, 16)] = rc_v[pl.ds(t, 16)] + fr_v[pl.ds(t, 16)]
            nx_v[pl.ds(t, 16)] = zeros16

    hop()
    hop()

    def fin_scan(base, with_outputs):
        @plsc.parallel_loop(0, half, 16, unroll=4)
        def _fin(e):
            e = pl.multiple_of(e, 16)
            s = src_v[pl.ds(base + e, 16)]
            d = dst_v[pl.ds(base + e, 16)]
            rs = plsc.load_gather(rc_v, [s])
            rd = plsc.load_gather(rc_v, [d])
            m = (rs > 0.0) & (rd > 0.0)
            plsc.store_scatter(ni_v, [s], ones16, mask=m)
            plsc.store_scatter(ni_v, [d], ones16, mask=m)
            if with_outputs:
                ls = plsc.load_gather(lg_v, [s])
                ld = plsc.load_gather(lg_v, [d])
                out_v[pl.ds(e, 16)] = jnp.where(m, ls + ld, 0.0)
                out_v[pl.ds(half + e, 16)] = jnp.where(m, ones16, zeros16)

    base_mine = pl.multiple_of(cid * half, 16)
    base_other = pl.multiple_of((1 - cid) * half, 16)
    fin_scan(base_mine, True)
    pltpu.sync_copy(out_v.at[pl.ds(0, half)],
                    ew_hbm.at[pl.ds(sid * EP + base_mine, half)])
    pltpu.sync_copy(out_v.at[pl.ds(half, half)],
                    em_hbm.at[pl.ds(sid * EP + base_mine, half)])
    fin_scan(base_other, False)

    combine(ni_v)

    @pl.when(cid == 0)
    def _wr_ni():
        pltpu.sync_copy(out_v.at[pl.ds(0, _SLC)],
                        ni_hbm.at[pl.ds(sid * _SLC, _SLC)])


def _edge_call(src, dst, seed_pad, logits_pad):
    E = src.shape[0]
    EP = E // 16
    mesh = plsc.VectorSubcoreMesh(core_axis_name="c", subcore_axis_name="s",
                                  num_cores=2, num_subcores=16)
    f32 = jnp.float32
    kern = functools.partial(
        pl.kernel,
        out_type=[jax.ShapeDtypeStruct((E,), f32),
                  jax.ShapeDtypeStruct((E,), f32),
                  jax.ShapeDtypeStruct((_NPAD,), f32)],
        mesh=mesh,
        compiler_params=pltpu.CompilerParams(needs_layout_passes=False),
        scratch_types=[
            pltpu.VMEM((EP,), jnp.int32),
            pltpu.VMEM((EP,), jnp.int32),
            pltpu.VMEM((EP,), f32),
            pltpu.VMEM((_NPAD,), f32),
            pltpu.VMEM((_NPAD,), f32),
            pltpu.VMEM((_NPAD,), f32),
            pltpu.VMEM((_NPAD,), f32),
            pltpu.VMEM((_NPAD,), f32),
            pltpu.VMEM_SHARED((16 * _NPAD,), f32),
            pltpu.VMEM_SHARED((_NPAD,), f32),
        ],
    )(functools.partial(_edge_body, EP))
    return kern(src, dst, seed_pad, logits_pad)


def kernel(h, g, batch_ids, edge_index, W1, b1, ln_g, ln_b, W2, b2):
    N = h.shape[0]
    logits, node_prob, seed = _head_call(h, g, batch_ids, W1, b1,
                                         ln_g, ln_b, W2, b2)
    src = edge_index[0].astype(jnp.int32)
    dst = edge_index[1].astype(jnp.int32)
    seed_pad = jnp.zeros((_NPAD,), jnp.float32).at[:N].set(seed)
    logits_pad = jnp.zeros((_NPAD,), jnp.float32).at[:N].set(logits)
    ew, emf, nic = _edge_call(src, dst, seed_pad, logits_pad)
    edge_mask = emf > 0.0
    node_in_mask = nic[:N] > 0.0
    return ew, node_prob, edge_mask, node_in_mask


# speedup vs baseline: 148.1421x; 1.1007x over previous
import functools

import jax
import jax.numpy as jnp
from jax import lax
from jax.experimental import pallas as pl
from jax.experimental.pallas import tpu as pltpu
from jax.experimental.pallas import tpu_sc as plsc

NEG = -1e30
BIG = 2**30


def _head_body(h_ref, bi_ref, g_ref, w1_ref, b1_ref, lng_ref, lnb_ref,
               w2_ref, b2_ref, logits_ref, prob_ref, seed_ref):
    nblk = pl.num_programs(0)
    i = pl.program_id(0)
    R = h_ref.shape[0]
    GB = g_ref.shape[0]

    bi = bi_ref[pl.ds(i, 1)].reshape(1, R)
    one = (bi == lax.broadcasted_iota(jnp.int32, (GB, R), 0))
    onef = one.astype(jnp.float32)
    g_rep = lax.dot_general(onef, g_ref[...], (((0,), (0,)), ((), ())),
                            preferred_element_type=jnp.float32)
    z = h_ref[...] + g_rep
    u = jnp.dot(z, w1_ref[...], preferred_element_type=jnp.float32) + b1_ref[...]
    mu = jnp.mean(u, axis=-1, keepdims=True)
    var = jnp.mean((u - mu) * (u - mu), axis=-1, keepdims=True)
    un = (u - mu) / jnp.sqrt(var + 1e-5) * lng_ref[...] + lnb_ref[...]
    ur = jnp.maximum(un, 0.0)
    lrow = lax.dot_general(w2_ref[...], ur, (((1,), (1,)), ((), ())),
                           preferred_element_type=jnp.float32) + b2_ref[...]
    logits_ref[pl.ds(i, 1)] = lrow.reshape(1, 1, R)

    @pl.when(i == nblk - 1)
    def _segment_stage():
        def chunk(c):
            l = logits_ref[pl.ds(c, 1)].reshape(1, R)
            b = bi_ref[pl.ds(c, 1)].reshape(1, R)
            o = (b == lax.broadcasted_iota(jnp.int32, (GB, R), 0))
            return l, o, o.astype(jnp.float32)

        def p1(c, mx):
            l, o, _ = chunk(c)
            return jnp.maximum(mx, jnp.max(jnp.where(o, l, NEG), axis=1,
                                           keepdims=True))
        mx = lax.fori_loop(0, nblk, p1, jnp.full((GB, 1), NEG, jnp.float32))

        def p2(c, ss):
            l, o, of = chunk(c)
            mxr = lax.dot_general(mx, of, (((0,), (0,)), ((), ())),
                                  preferred_element_type=jnp.float32)
            ex = jnp.exp(l - mxr)
            return ss + jnp.sum(jnp.where(o, ex, 0.0), axis=1, keepdims=True)
        ss = lax.fori_loop(0, nblk, p2, jnp.zeros((GB, 1), jnp.float32))

        def p3(c, pm):
            l, o, of = chunk(c)
            mxr = lax.dot_general(mx, of, (((0,), (0,)), ((), ())),
                                  preferred_element_type=jnp.float32)
            ssr = lax.dot_general(ss, of, (((0,), (0,)), ((), ())),
                                  preferred_element_type=jnp.float32)
            p = jnp.exp(l - mxr) / ssr
            prob_ref[pl.ds(c, 1)] = p.reshape(1, 1, R)
            return jnp.maximum(pm, jnp.max(jnp.where(o, p, NEG), axis=1,
                                           keepdims=True))
        pm = lax.fori_loop(0, nblk, p3, jnp.full((GB, 1), NEG, jnp.float32))

        def p4(c, im):
            _, o, _ = chunk(c)
            p = prob_ref[pl.ds(c, 1)].reshape(1, R)
            gidx = c * R + lax.broadcasted_iota(jnp.int32, (1, R), 1)
            cand = jnp.where(o & (p == pm), gidx, BIG)
            return jnp.minimum(im, jnp.min(cand, axis=1, keepdims=True))
        im = lax.fori_loop(0, nblk, p4, jnp.full((GB, 1), BIG, jnp.int32))

        def p5(c, _):
            _, o, _ = chunk(c)
            gidx = c * R + lax.broadcasted_iota(jnp.int32, (1, R), 1)
            hit = jnp.where(o & (gidx == im), 1.0, 0.0)
            seed_ref[pl.ds(c, 1)] = jnp.max(hit, axis=0, keepdims=True
                                            ).reshape(1, 1, R)
            return 0
        lax.fori_loop(0, nblk, p5, 0)


def _head_call(h, g, batch_ids, W1, b1, ln_g, ln_b, W2, b2):
    N, HID = h.shape
    B = g.shape[0]
    H2 = W1.shape[1]
    R = 1000
    nblk = N // R
    GB = 128
    g_pad = jnp.zeros((GB, HID), jnp.float32).at[:B].set(g)
    bi3 = batch_ids.astype(jnp.int32).reshape(nblk, 1, R)

    full3 = lambda s: pl.BlockSpec(s, lambda i: (0, 0, 0))
    full2 = lambda s: pl.BlockSpec(s, lambda i: (0, 0))
    outs = pl.pallas_call(
        _head_body,
        grid=(nblk,),
        in_specs=[
            pl.BlockSpec((R, HID), lambda i: (i, 0)),
            full3((nblk, 1, R)),
            full2((GB, HID)),
            full2((HID, H2)),
            full2((1, H2)),
            full2((1, H2)),
            full2((1, H2)),
            full2((1, H2)),
            full2((1, 1)),
        ],
        out_specs=[full3((nblk, 1, R)), full3((nblk, 1, R)),
                   full3((nblk, 1, R))],
        out_shape=[jax.ShapeDtypeStruct((nblk, 1, R), jnp.float32)] * 3,
    )(h, bi3, g_pad, W1, b1.reshape(1, H2), ln_g.reshape(1, H2),
      ln_b.reshape(1, H2), W2.reshape(1, H2), b2.reshape(1, 1))
    logits3, prob3, seed3 = outs
    return logits3.reshape(N), prob3.reshape(N), seed3.reshape(N)


_NPAD = 10240
_SLC = _NPAD // 16


def _edge_body(EP, N, edge_hbm, seed_hbm, logits_hbm,
               ew_hbm, em_hbm, ni_hbm,
               src_v, dst_v, out_v, fr_v, nx_v, rc_v, lg_v, ni_v,
               sh_all, sh_comb):
    sid = lax.axis_index("s")
    cid = lax.axis_index("c")
    E = EP * 16
    half = EP // 2
    zeros16 = jnp.zeros((16,), jnp.float32)
    ones16 = jnp.ones((16,), jnp.float32)

    pltpu.sync_copy(edge_hbm.at[pl.ds(sid * EP, EP)], src_v)
    pltpu.sync_copy(edge_hbm.at[pl.ds(E + sid * EP, EP)], dst_v)
    pltpu.sync_copy(seed_hbm, fr_v.at[pl.ds(0, N)])
    pltpu.sync_copy(seed_hbm, rc_v.at[pl.ds(0, N)])
    pltpu.sync_copy(logits_hbm, lg_v.at[pl.ds(0, N)])

    @plsc.parallel_loop(0, _NPAD, 16, unroll=8)
    def _zero(t):
        t = pl.multiple_of(t, 16)
        nx_v[pl.ds(t, 16)] = zeros16
        ni_v[pl.ds(t, 16)] = zeros16

    @plsc.parallel_loop(N, _NPAD, 16)
    def _zero_tail(t):
        t = pl.multiple_of(t, 16)
        fr_v[pl.ds(t, 16)] = zeros16
        rc_v[pl.ds(t, 16)] = zeros16

    def combine(part_v):
        pltpu.sync_copy(part_v, sh_all.at[pl.ds(sid * _NPAD, _NPAD)])
        plsc.subcore_barrier()
        for c in range(16):
            pltpu.sync_copy(sh_all.at[pl.ds(c * _NPAD + sid * _SLC, _SLC)],
                            out_v.at[pl.ds(c * _SLC, _SLC)])

        @plsc.parallel_loop(0, _SLC, 16, unroll=4)
        def _sum(t):
            t = pl.multiple_of(t, 16)
            acc = out_v[pl.ds(t, 16)]
            for c in range(1, 16):
                acc = acc + out_v[pl.ds(c * _SLC + t, 16)]
            out_v[pl.ds(t, 16)] = acc

    def hop():
        @plsc.parallel_loop(0, EP, 16, unroll=8)
        def _scan(e):
            e = pl.multiple_of(e, 16)
            d = dst_v[pl.ds(e, 16)]
            fm = plsc.load_gather(fr_v, [d])
            s = src_v[pl.ds(e, 16)]
            plsc.store_scatter(nx_v, [s], ones16, mask=fm > 0.0)

        combine(nx_v)
        pltpu.sync_copy(out_v.at[pl.ds(0, _SLC)],
                        sh_comb.at[pl.ds(sid * _SLC, _SLC)])
        plsc.subcore_barrier()
        pltpu.sync_copy(sh_comb, fr_v)

        @plsc.parallel_loop(0, _NPAD, 16, unroll=8)
        def _upd(t):
            t = pl.multiple_of(t, 16)
            rc_v[pl.ds(t, 16)] = rc_v[pl.ds(t, 16)] + fr_v[pl.ds(t, 16)]
            nx_v[pl.ds(t, 16)] = zeros16

    hop()
    hop()

    def fin_scan(base, with_outputs):
        @plsc.parallel_loop(0, half, 16, unroll=4)
        def _fin(e):
            e = pl.multiple_of(e, 16)
            s = src_v[pl.ds(base + e, 16)]
            d = dst_v[pl.ds(base + e, 16)]
            rs = plsc.load_gather(rc_v, [s])
            rd = plsc.load_gather(rc_v, [d])
            m = (rs > 0.0) & (rd > 0.0)
            plsc.store_scatter(ni_v, [s], ones16, mask=m)
            plsc.store_scatter(ni_v, [d], ones16, mask=m)
            if with_outputs:
                ls = plsc.load_gather(lg_v, [s])
                ld = plsc.load_gather(lg_v, [d])
                out_v[pl.ds(e, 16)] = jnp.where(m, ls + ld, 0.0)
                out_v[pl.ds(half + e, 16)] = jnp.where(m, ones16, zeros16)

    base_mine = pl.multiple_of(cid * half, 16)
    base_other = pl.multiple_of((1 - cid) * half, 16)
    fin_scan(base_mine, True)
    pltpu.sync_copy(out_v.at[pl.ds(0, half)],
                    ew_hbm.at[pl.ds(sid * EP + base_mine, half)])
    pltpu.sync_copy(out_v.at[pl.ds(half, half)],
                    em_hbm.at[pl.ds(sid * EP + base_mine, half)])
    fin_scan(base_other, False)

    combine(ni_v)

    @pl.when(cid == 0)
    def _wr_ni():
        pltpu.sync_copy(out_v.at[pl.ds(0, _SLC)],
                        ni_hbm.at[pl.ds(sid * _SLC, _SLC)])


def _edge_call(edge_flat, seed, logits):
    E = edge_flat.shape[0] // 2
    N = seed.shape[0]
    EP = E // 16
    mesh = plsc.VectorSubcoreMesh(core_axis_name="c", subcore_axis_name="s",
                                  num_cores=2, num_subcores=16)
    f32 = jnp.float32
    kern = functools.partial(
        pl.kernel,
        out_type=[jax.ShapeDtypeStruct((E,), f32),
                  jax.ShapeDtypeStruct((E,), f32),
                  jax.ShapeDtypeStruct((_NPAD,), f32)],
        mesh=mesh,
        compiler_params=pltpu.CompilerParams(needs_layout_passes=False),
        scratch_types=[
            pltpu.VMEM((EP,), jnp.int32),
            pltpu.VMEM((EP,), jnp.int32),
            pltpu.VMEM((EP,), f32),
            pltpu.VMEM((_NPAD,), f32),
            pltpu.VMEM((_NPAD,), f32),
            pltpu.VMEM((_NPAD,), f32),
            pltpu.VMEM((_NPAD,), f32),
            pltpu.VMEM((_NPAD,), f32),
            pltpu.VMEM_SHARED((16 * _NPAD,), f32),
            pltpu.VMEM_SHARED((_NPAD,), f32),
        ],
    )(functools.partial(_edge_body, EP, N))
    return kern(edge_flat, seed, logits)


def kernel(h, g, batch_ids, edge_index, W1, b1, ln_g, ln_b, W2, b2):
    N = h.shape[0]
    logits, node_prob, seed = _head_call(h, g, batch_ids, W1, b1,
                                         ln_g, ln_b, W2, b2)
    edge_flat = edge_index.astype(jnp.int32).reshape(-1)
    ew, emf, nic = _edge_call(edge_flat, seed, logits)
    edge_mask = emf > 0.0
    node_in_mask = nic[:N] > 0.0
    return ew, node_prob, edge_mask, node_in_mask


# speedup vs baseline: 168.5300x; 1.1376x over previous
import functools

import jax
import jax.numpy as jnp
from jax import lax
from jax.experimental import pallas as pl
from jax.experimental.pallas import tpu as pltpu
from jax.experimental.pallas import tpu_sc as plsc

NEG = -1e30
BIG = 2**30


def _head_body(h_ref, bi_ref, g_ref, w1_ref, b1_ref, lng_ref, lnb_ref,
               w2_ref, b2_ref, logits_ref, prob_ref, seed_ref):
    nblk = pl.num_programs(0)
    i = pl.program_id(0)
    R = h_ref.shape[0]
    GB = g_ref.shape[0]

    bi = bi_ref[pl.ds(i, 1)].reshape(1, R)
    one = (bi == lax.broadcasted_iota(jnp.int32, (GB, R), 0))
    onef = one.astype(jnp.float32)
    g_rep = lax.dot_general(onef, g_ref[...], (((0,), (0,)), ((), ())),
                            preferred_element_type=jnp.float32)
    z = h_ref[...] + g_rep
    u = jnp.dot(z, w1_ref[...], preferred_element_type=jnp.float32) + b1_ref[...]
    H2 = u.shape[1]
    mu = jnp.sum(u, axis=-1, keepdims=True) * (1.0 / H2)
    d = u - mu
    var = jnp.sum(d * d, axis=-1, keepdims=True) * (1.0 / H2)
    un = d * lax.rsqrt(var + 1e-5) * lng_ref[...] + lnb_ref[...]
    ur = jnp.maximum(un, 0.0)
    lrow = lax.dot_general(w2_ref[...], ur, (((1,), (1,)), ((), ())),
                           preferred_element_type=jnp.float32) + b2_ref[...]
    logits_ref[pl.ds(i, 1)] = lrow.reshape(1, 1, R)

    @pl.when(i == nblk - 1)
    def _segment_stage():
        def chunk(c):
            l = logits_ref[c].reshape(1, R)
            b = bi_ref[c].reshape(1, R)
            o = (b == lax.broadcasted_iota(jnp.int32, (GB, R), 0))
            return l, o, o.astype(jnp.float32)

        mx = jnp.full((GB, 1), NEG, jnp.float32)
        for c in range(nblk):
            l, o, _ = chunk(c)
            mx = jnp.maximum(mx, jnp.max(jnp.where(o, l, NEG), axis=1,
                                         keepdims=True))

        ss = jnp.zeros((GB, 1), jnp.float32)
        for c in range(nblk):
            l, o, of = chunk(c)
            mxr = lax.dot_general(mx, of, (((0,), (0,)), ((), ())),
                                  preferred_element_type=jnp.float32)
            ex = jnp.exp(l - mxr)
            ss = ss + jnp.sum(jnp.where(o, ex, 0.0), axis=1, keepdims=True)

        pm = jnp.full((GB, 1), NEG, jnp.float32)
        for c in range(nblk):
            l, o, of = chunk(c)
            mxr = lax.dot_general(mx, of, (((0,), (0,)), ((), ())),
                                  preferred_element_type=jnp.float32)
            ssr = lax.dot_general(ss, of, (((0,), (0,)), ((), ())),
                                  preferred_element_type=jnp.float32)
            p = jnp.exp(l - mxr) / ssr
            prob_ref[c] = p.reshape(1, R)
            pm = jnp.maximum(pm, jnp.max(jnp.where(o, p, NEG), axis=1,
                                         keepdims=True))

        im = jnp.full((GB, 1), BIG, jnp.int32)
        for c in range(nblk):
            _, o, _ = chunk(c)
            p = prob_ref[c].reshape(1, R)
            gidx = c * R + lax.broadcasted_iota(jnp.int32, (1, R), 1)
            cand = jnp.where(o & (p == pm), gidx, BIG)
            im = jnp.minimum(im, jnp.min(cand, axis=1, keepdims=True))

        for c in range(nblk):
            _, o, _ = chunk(c)
            gidx = c * R + lax.broadcasted_iota(jnp.int32, (1, R), 1)
            hit = jnp.where(o & (gidx == im), 1.0, 0.0)
            seed_ref[c] = jnp.max(hit, axis=0, keepdims=True).reshape(1, R)


def _head_call(h, g, batch_ids, W1, b1, ln_g, ln_b, W2, b2):
    N, HID = h.shape
    B = g.shape[0]
    H2 = W1.shape[1]
    R = 2000
    nblk = N // R
    GB = 128
    g_pad = jnp.zeros((GB, HID), jnp.float32).at[:B].set(g)
    bi3 = batch_ids.astype(jnp.int32).reshape(nblk, 1, R)

    full3 = lambda s: pl.BlockSpec(s, lambda i: (0, 0, 0))
    full2 = lambda s: pl.BlockSpec(s, lambda i: (0, 0))
    outs = pl.pallas_call(
        _head_body,
        grid=(nblk,),
        in_specs=[
            pl.BlockSpec((R, HID), lambda i: (i, 0)),
            full3((nblk, 1, R)),
            full2((GB, HID)),
            full2((HID, H2)),
            full2((1, H2)),
            full2((1, H2)),
            full2((1, H2)),
            full2((1, H2)),
            full2((1, 1)),
        ],
        out_specs=[full3((nblk, 1, R)), full3((nblk, 1, R)),
                   full3((nblk, 1, R))],
        out_shape=[jax.ShapeDtypeStruct((nblk, 1, R), jnp.float32)] * 3,
    )(h, bi3, g_pad, W1, b1.reshape(1, H2), ln_g.reshape(1, H2),
      ln_b.reshape(1, H2), W2.reshape(1, H2), b2.reshape(1, 1))
    logits3, prob3, seed3 = outs
    return logits3.reshape(N), prob3.reshape(N), seed3.reshape(N)


_NPAD = 10240
_SLC = _NPAD // 16


def _edge_body(EP, N, edge_hbm, seed_hbm, logits_hbm,
               ew_hbm, em_hbm, ni_hbm,
               src_v, dst_v, out_v, fr_v, nx_v, rc_v, lg_v, ni_v,
               sh_all, sh_comb):
    sid = lax.axis_index("s")
    cid = lax.axis_index("c")
    E = EP * 16
    half = EP // 2
    zeros16 = jnp.zeros((16,), jnp.float32)
    ones16 = jnp.ones((16,), jnp.float32)

    pltpu.sync_copy(edge_hbm.at[pl.ds(sid * EP, EP)], src_v)
    pltpu.sync_copy(edge_hbm.at[pl.ds(E + sid * EP, EP)], dst_v)
    pltpu.sync_copy(seed_hbm, fr_v.at[pl.ds(0, N)])
    pltpu.sync_copy(seed_hbm, rc_v.at[pl.ds(0, N)])
    pltpu.sync_copy(logits_hbm, lg_v.at[pl.ds(0, N)])

    @plsc.parallel_loop(0, _NPAD, 16, unroll=8)
    def _zero(t):
        t = pl.multiple_of(t, 16)
        nx_v[pl.ds(t, 16)] = zeros16
        ni_v[pl.ds(t, 16)] = zeros16

    @plsc.parallel_loop(N, _NPAD, 16)
    def _zero_tail(t):
        t = pl.multiple_of(t, 16)
        fr_v[pl.ds(t, 16)] = zeros16
        rc_v[pl.ds(t, 16)] = zeros16

    def combine(part_v):
        pltpu.sync_copy(part_v, sh_all.at[pl.ds(sid * _NPAD, _NPAD)])
        plsc.subcore_barrier()
        for c in range(16):
            pltpu.sync_copy(sh_all.at[pl.ds(c * _NPAD + sid * _SLC, _SLC)],
                            out_v.at[pl.ds(c * _SLC, _SLC)])

        @plsc.parallel_loop(0, _SLC, 16, unroll=4)
        def _sum(t):
            t = pl.multiple_of(t, 16)
            acc = out_v[pl.ds(t, 16)]
            for c in range(1, 16):
                acc = acc + out_v[pl.ds(c * _SLC + t, 16)]
            out_v[pl.ds(t, 16)] = acc

    def hop():
        @plsc.parallel_loop(0, EP, 16, unroll=8)
        def _scan(e):
            e = pl.multiple_of(e, 16)
            d = dst_v[pl.ds(e, 16)]
            fm = plsc.load_gather(fr_v, [d])
            s = src_v[pl.ds(e, 16)]
            plsc.store_scatter(nx_v, [s], ones16, mask=fm > 0.0)

        combine(nx_v)
        pltpu.sync_copy(out_v.at[pl.ds(0, _SLC)],
                        sh_comb.at[pl.ds(sid * _SLC, _SLC)])
        plsc.subcore_barrier()
        pltpu.sync_copy(sh_comb, fr_v)

        @plsc.parallel_loop(0, _NPAD, 16, unroll=8)
        def _upd(t):
            t = pl.multiple_of(t, 16)
            rc_v[pl.ds(t, 16)] = rc_v[pl.ds(t, 16)] + fr_v[pl.ds(t, 16)]
            nx_v[pl.ds(t, 16)] = zeros16

    hop()
    hop()

    def fin_scan(base, with_outputs):
        @plsc.parallel_loop(0, half, 16, unroll=4)
        def _fin(e):
            e = pl.multiple_of(e, 16)
            s = src_v[pl.ds(base + e, 16)]
            d = dst_v[pl.ds(base + e, 16)]
            rs = plsc.load_gather(rc_v, [s])
            rd = plsc.load_gather(rc_v, [d])
            m = (rs > 0.0) & (rd > 0.0)
            plsc.store_scatter(ni_v, [s], ones16, mask=m)
            plsc.store_scatter(ni_v, [d], ones16, mask=m)
            if with_outputs:
                ls = plsc.load_gather(lg_v, [s])
                ld = plsc.load_gather(lg_v, [d])
                out_v[pl.ds(e, 16)] = jnp.where(m, ls + ld, 0.0)
                out_v[pl.ds(half + e, 16)] = jnp.where(m, ones16, zeros16)

    base_mine = pl.multiple_of(cid * half, 16)
    base_other = pl.multiple_of((1 - cid) * half, 16)
    fin_scan(base_mine, True)
    pltpu.sync_copy(out_v.at[pl.ds(0, half)],
                    ew_hbm.at[pl.ds(sid * EP + base_mine, half)])
    pltpu.sync_copy(out_v.at[pl.ds(half, half)],
                    em_hbm.at[pl.ds(sid * EP + base_mine, half)])
    fin_scan(base_other, False)

    combine(ni_v)

    @pl.when(cid == 0)
    def _wr_ni():
        pltpu.sync_copy(out_v.at[pl.ds(0, _SLC)],
                        ni_hbm.at[pl.ds(sid * _SLC, _SLC)])


def _edge_call(edge_flat, seed, logits):
    E = edge_flat.shape[0] // 2
    N = seed.shape[0]
    EP = E // 16
    mesh = plsc.VectorSubcoreMesh(core_axis_name="c", subcore_axis_name="s",
                                  num_cores=2, num_subcores=16)
    f32 = jnp.float32
    kern = functools.partial(
        pl.kernel,
        out_type=[jax.ShapeDtypeStruct((E,), f32),
                  jax.ShapeDtypeStruct((E,), f32),
                  jax.ShapeDtypeStruct((_NPAD,), f32)],
        mesh=mesh,
        compiler_params=pltpu.CompilerParams(needs_layout_passes=False),
        scratch_types=[
            pltpu.VMEM((EP,), jnp.int32),
            pltpu.VMEM((EP,), jnp.int32),
            pltpu.VMEM((EP,), f32),
            pltpu.VMEM((_NPAD,), f32),
            pltpu.VMEM((_NPAD,), f32),
            pltpu.VMEM((_NPAD,), f32),
            pltpu.VMEM((_NPAD,), f32),
            pltpu.VMEM((_NPAD,), f32),
            pltpu.VMEM_SHARED((16 * _NPAD,), f32),
            pltpu.VMEM_SHARED((_NPAD,), f32),
        ],
    )(functools.partial(_edge_body, EP, N))
    return kern(edge_flat, seed, logits)


def kernel(h, g, batch_ids, edge_index, W1, b1, ln_g, ln_b, W2, b2):
    N = h.shape[0]
    logits, node_prob, seed = _head_call(h, g, batch_ids, W1, b1,
                                         ln_g, ln_b, W2, b2)
    edge_flat = edge_index.astype(jnp.int32).reshape(-1)
    ew, emf, nic = _edge_call(edge_flat, seed, logits)
    edge_mask = emf > 0.0
    node_in_mask = nic[:N] > 0.0
    return ew, node_prob, edge_mask, node_in_mask


# speedup vs baseline: 178.0948x; 1.0568x over previous
import functools

import jax
import jax.numpy as jnp
from jax import lax
from jax.experimental import pallas as pl
from jax.experimental.pallas import tpu as pltpu
from jax.experimental.pallas import tpu_sc as plsc

NEG = -1e30
BIG = 2**30


def _head_body(h_ref, bi_ref, g_ref, w1_ref, b1_ref, lng_ref, lnb_ref,
               w2_ref, b2_ref, logits_ref, prob_ref, seed_ref):
    nblk = pl.num_programs(0)
    i = pl.program_id(0)
    R = h_ref.shape[0]
    GB = g_ref.shape[0]

    bi = bi_ref[pl.ds(i, 1)].reshape(1, R)
    one = (bi == lax.broadcasted_iota(jnp.int32, (GB, R), 0))
    onef = one.astype(jnp.float32)
    g_rep = lax.dot_general(onef, g_ref[...], (((0,), (0,)), ((), ())),
                            preferred_element_type=jnp.float32)
    z = h_ref[...] + g_rep
    u = jnp.dot(z, w1_ref[...], preferred_element_type=jnp.float32) + b1_ref[...]
    H2 = u.shape[1]
    mu = jnp.sum(u, axis=-1, keepdims=True) * (1.0 / H2)
    d = u - mu
    var = jnp.sum(d * d, axis=-1, keepdims=True) * (1.0 / H2)
    un = d * lax.rsqrt(var + 1e-5) * lng_ref[...] + lnb_ref[...]
    ur = jnp.maximum(un, 0.0)
    lrow = lax.dot_general(w2_ref[...], ur, (((1,), (1,)), ((), ())),
                           preferred_element_type=jnp.float32) + b2_ref[...]
    logits_ref[pl.ds(i, 1)] = lrow.reshape(1, 1, R)

    @pl.when(i == nblk - 1)
    def _segment_stage():
        def chunk(c):
            l = logits_ref[c].reshape(1, R)
            b = bi_ref[c].reshape(1, R)
            o = (b == lax.broadcasted_iota(jnp.int32, (GB, R), 0))
            return l, o, o.astype(jnp.float32)

        mx = jnp.full((GB, 1), NEG, jnp.float32)
        for c in range(nblk):
            l, o, _ = chunk(c)
            mx = jnp.maximum(mx, jnp.max(jnp.where(o, l, NEG), axis=1,
                                         keepdims=True))

        ss = jnp.zeros((GB, 1), jnp.float32)
        for c in range(nblk):
            l, o, of = chunk(c)
            mxr = lax.dot_general(mx, of, (((0,), (0,)), ((), ())),
                                  preferred_element_type=jnp.float32)
            ex = jnp.exp(l - mxr)
            ss = ss + jnp.sum(jnp.where(o, ex, 0.0), axis=1, keepdims=True)

        pm = jnp.full((GB, 1), NEG, jnp.float32)
        for c in range(nblk):
            l, o, of = chunk(c)
            mxr = lax.dot_general(mx, of, (((0,), (0,)), ((), ())),
                                  preferred_element_type=jnp.float32)
            ssr = lax.dot_general(ss, of, (((0,), (0,)), ((), ())),
                                  preferred_element_type=jnp.float32)
            p = jnp.exp(l - mxr) / ssr
            prob_ref[c] = p.reshape(1, R)
            pm = jnp.maximum(pm, jnp.max(jnp.where(o, p, NEG), axis=1,
                                         keepdims=True))

        im = jnp.full((GB, 1), BIG, jnp.int32)
        for c in range(nblk):
            _, o, _ = chunk(c)
            p = prob_ref[c].reshape(1, R)
            gidx = c * R + lax.broadcasted_iota(jnp.int32, (1, R), 1)
            cand = jnp.where(o & (p == pm), gidx, BIG)
            im = jnp.minimum(im, jnp.min(cand, axis=1, keepdims=True))

        for c in range(nblk):
            _, o, _ = chunk(c)
            gidx = c * R + lax.broadcasted_iota(jnp.int32, (1, R), 1)
            hit = jnp.where(o & (gidx == im), 1.0, 0.0)
            seed_ref[c] = jnp.max(hit, axis=0, keepdims=True).reshape(1, R)


def _head_call(h, g, batch_ids, W1, b1, ln_g, ln_b, W2, b2):
    N, HID = h.shape
    B = g.shape[0]
    H2 = W1.shape[1]
    R = 2000
    nblk = N // R
    GB = 128
    g_pad = jnp.zeros((GB, HID), jnp.float32).at[:B].set(g)
    bi3 = batch_ids.astype(jnp.int32).reshape(nblk, 1, R)

    full3 = lambda s: pl.BlockSpec(s, lambda i: (0, 0, 0))
    full2 = lambda s: pl.BlockSpec(s, lambda i: (0, 0))
    outs = pl.pallas_call(
        _head_body,
        grid=(nblk,),
        in_specs=[
            pl.BlockSpec((R, HID), lambda i: (i, 0)),
            full3((nblk, 1, R)),
            full2((GB, HID)),
            full2((HID, H2)),
            full2((1, H2)),
            full2((1, H2)),
            full2((1, H2)),
            full2((1, H2)),
            full2((1, 1)),
        ],
        out_specs=[full3((nblk, 1, R)), full3((nblk, 1, R)),
                   full3((nblk, 1, R))],
        out_shape=[jax.ShapeDtypeStruct((nblk, 1, R), jnp.float32)] * 3,
    )(h, bi3, g_pad, W1, b1.reshape(1, H2), ln_g.reshape(1, H2),
      ln_b.reshape(1, H2), W2.reshape(1, H2), b2.reshape(1, 1))
    logits3, prob3, seed3 = outs
    return logits3.reshape(N), prob3.reshape(N), seed3.reshape(N)


_NPAD = 10240
_SLC = _NPAD // 16


def _edge_body(EP, N, edge_hbm, seed_hbm, logits_hbm,
               ew_hbm, em_hbm, ni_hbm,
               src_v, dst_v, out_v, fr_v, nx_v, rc_v, lg_v, ni_v,
               sem, sh_all, sh_comb):
    sid = lax.axis_index("s")
    cid = lax.axis_index("c")
    E = EP * 16
    half = EP // 2
    zeros16 = jnp.zeros((16,), jnp.float32)
    ones16 = jnp.ones((16,), jnp.float32)

    pltpu.sync_copy(edge_hbm.at[pl.ds(sid * EP, EP)], src_v)
    pltpu.sync_copy(edge_hbm.at[pl.ds(E + sid * EP, EP)], dst_v)
    pltpu.sync_copy(seed_hbm, fr_v.at[pl.ds(0, N)])
    pltpu.sync_copy(seed_hbm, rc_v.at[pl.ds(0, N)])
    pltpu.sync_copy(logits_hbm, lg_v.at[pl.ds(0, N)])

    @plsc.parallel_loop(0, _NPAD, 16, unroll=8)
    def _zero(t):
        t = pl.multiple_of(t, 16)
        nx_v[pl.ds(t, 16)] = zeros16
        ni_v[pl.ds(t, 16)] = zeros16

    @plsc.parallel_loop(N, _NPAD, 16)
    def _zero_tail(t):
        t = pl.multiple_of(t, 16)
        fr_v[pl.ds(t, 16)] = zeros16
        rc_v[pl.ds(t, 16)] = zeros16

    def combine(part_v):
        pltpu.sync_copy(part_v, sh_all.at[pl.ds(sid * _NPAD, _NPAD)])
        plsc.subcore_barrier()
        rds = [pltpu.async_copy(
            sh_all.at[pl.ds(c * _NPAD + sid * _SLC, _SLC)],
            out_v.at[pl.ds(c * _SLC, _SLC)], sem) for c in range(16)]
        for cp in rds:
            cp.wait()

        @plsc.parallel_loop(0, _SLC, 16, unroll=4)
        def _sum(t):
            t = pl.multiple_of(t, 16)
            acc = out_v[pl.ds(t, 16)]
            for c in range(1, 16):
                acc = acc + out_v[pl.ds(c * _SLC + t, 16)]
            out_v[pl.ds(t, 16)] = acc

    def hop():
        @plsc.parallel_loop(0, EP, 16, unroll=8)
        def _scan(e):
            e = pl.multiple_of(e, 16)
            d = dst_v[pl.ds(e, 16)]
            fm = plsc.load_gather(fr_v, [d])
            s = src_v[pl.ds(e, 16)]
            plsc.store_scatter(nx_v, [s], ones16, mask=fm > 0.0)

        combine(nx_v)
        pltpu.sync_copy(out_v.at[pl.ds(0, _SLC)],
                        sh_comb.at[pl.ds(sid * _SLC, _SLC)])
        plsc.subcore_barrier()
        pltpu.sync_copy(sh_comb, fr_v)

        @plsc.parallel_loop(0, _NPAD, 16, unroll=8)
        def _upd(t):
            t = pl.multiple_of(t, 16)
            rc_v[pl.ds(t, 16)] = rc_v[pl.ds(t, 16)] + fr_v[pl.ds(t, 16)]
            nx_v[pl.ds(t, 16)] = zeros16

    hop()
    hop()

    def fin_scan(base, with_outputs):
        @plsc.parallel_loop(0, half, 16, unroll=4)
        def _fin(e):
            e = pl.multiple_of(e, 16)
            s = src_v[pl.ds(base + e, 16)]
            d = dst_v[pl.ds(base + e, 16)]
            rs = plsc.load_gather(rc_v, [s])
            rd = plsc.load_gather(rc_v, [d])
            m = (rs > 0.0) & (rd > 0.0)
            plsc.store_scatter(ni_v, [s], ones16, mask=m)
            plsc.store_scatter(ni_v, [d], ones16, mask=m)
            if with_outputs:
                ls = plsc.load_gather(lg_v, [s])
                ld = plsc.load_gather(lg_v, [d])
                out_v[pl.ds(e, 16)] = jnp.where(m, ls + ld, 0.0)
                out_v[pl.ds(half + e, 16)] = jnp.where(m, ones16, zeros16)

    base_mine = pl.multiple_of(cid * half, 16)
    base_other = pl.multiple_of((1 - cid) * half, 16)
    fin_scan(base_mine, True)
    pltpu.sync_copy(out_v.at[pl.ds(0, half)],
                    ew_hbm.at[pl.ds(sid * EP + base_mine, half)])
    pltpu.sync_copy(out_v.at[pl.ds(half, half)],
                    em_hbm.at[pl.ds(sid * EP + base_mine, half)])
    fin_scan(base_other, False)

    combine(ni_v)

    @pl.when(cid == 0)
    def _wr_ni():
        pltpu.sync_copy(out_v.at[pl.ds(0, _SLC)],
                        ni_hbm.at[pl.ds(sid * _SLC, _SLC)])


def _edge_call(edge_flat, seed, logits):
    E = edge_flat.shape[0] // 2
    N = seed.shape[0]
    EP = E // 16
    mesh = plsc.VectorSubcoreMesh(core_axis_name="c", subcore_axis_name="s",
                                  num_cores=2, num_subcores=16)
    f32 = jnp.float32
    kern = functools.partial(
        pl.kernel,
        out_type=[jax.ShapeDtypeStruct((E,), f32),
                  jax.ShapeDtypeStruct((E,), f32),
                  jax.ShapeDtypeStruct((_NPAD,), f32)],
        mesh=mesh,
        compiler_params=pltpu.CompilerParams(needs_layout_passes=False),
        scratch_types=[
            pltpu.VMEM((EP,), jnp.int32),
            pltpu.VMEM((EP,), jnp.int32),
            pltpu.VMEM((EP,), f32),
            pltpu.VMEM((_NPAD,), f32),
            pltpu.VMEM((_NPAD,), f32),
            pltpu.VMEM((_NPAD,), f32),
            pltpu.VMEM((_NPAD,), f32),
            pltpu.VMEM((_NPAD,), f32),
            pltpu.SemaphoreType.DMA,
            pltpu.VMEM_SHARED((16 * _NPAD,), f32),
            pltpu.VMEM_SHARED((_NPAD,), f32),
        ],
    )(functools.partial(_edge_body, EP, N))
    return kern(edge_flat, seed, logits)


def kernel(h, g, batch_ids, edge_index, W1, b1, ln_g, ln_b, W2, b2):
    N = h.shape[0]
    logits, node_prob, seed = _head_call(h, g, batch_ids, W1, b1,
                                         ln_g, ln_b, W2, b2)
    edge_flat = edge_index.astype(jnp.int32).reshape(-1)
    ew, emf, nic = _edge_call(edge_flat, seed, logits)
    edge_mask = emf > 0.0
    node_in_mask = nic[:N] > 0.0
    return ew, node_prob, edge_mask, node_in_mask


# speedup vs baseline: 183.5832x; 1.0308x over previous
import functools

import jax
import jax.numpy as jnp
from jax import lax
from jax.experimental import pallas as pl
from jax.experimental.pallas import tpu as pltpu
from jax.experimental.pallas import tpu_sc as plsc

NEG = -1e30
BIG = 2**30


def _head_body(h_ref, bi_ref, g_ref, w1_ref, b1_ref, lng_ref, lnb_ref,
               w2_ref, b2_ref, logits_ref, prob_ref, seed_ref):
    nblk = pl.num_programs(0)
    i = pl.program_id(0)
    R = h_ref.shape[0]
    GB = g_ref.shape[0]

    bi = bi_ref[pl.ds(i, 1)].reshape(1, R)
    one = (bi == lax.broadcasted_iota(jnp.int32, (GB, R), 0))
    onef = one.astype(jnp.float32)
    g_rep = lax.dot_general(onef, g_ref[...], (((0,), (0,)), ((), ())),
                            preferred_element_type=jnp.float32)
    z = h_ref[...] + g_rep
    u = jnp.dot(z, w1_ref[...], preferred_element_type=jnp.float32) + b1_ref[...]
    H2 = u.shape[1]
    mu = jnp.sum(u, axis=-1, keepdims=True) * (1.0 / H2)
    d = u - mu
    var = jnp.sum(d * d, axis=-1, keepdims=True) * (1.0 / H2)
    un = d * lax.rsqrt(var + 1e-5) * lng_ref[...] + lnb_ref[...]
    ur = jnp.maximum(un, 0.0)
    lrow = lax.dot_general(w2_ref[...], ur, (((1,), (1,)), ((), ())),
                           preferred_element_type=jnp.float32) + b2_ref[...]
    logits_ref[pl.ds(i, 1)] = lrow.reshape(1, 1, R)

    @pl.when(i == nblk - 1)
    def _segment_stage():
        def chunk(c):
            l = logits_ref[c].reshape(1, R)
            b = bi_ref[c].reshape(1, R)
            o = (b == lax.broadcasted_iota(jnp.int32, (GB, R), 0))
            return l, o, o.astype(jnp.float32)

        mx = jnp.full((GB, 1), NEG, jnp.float32)
        for c in range(nblk):
            l, o, _ = chunk(c)
            mx = jnp.maximum(mx, jnp.max(jnp.where(o, l, NEG), axis=1,
                                         keepdims=True))

        ss = jnp.zeros((GB, 1), jnp.float32)
        for c in range(nblk):
            l, o, of = chunk(c)
            mxr = lax.dot_general(mx, of, (((0,), (0,)), ((), ())),
                                  preferred_element_type=jnp.float32)
            ex = jnp.exp(l - mxr)
            ss = ss + jnp.sum(jnp.where(o, ex, 0.0), axis=1, keepdims=True)

        pm = jnp.full((GB, 1), NEG, jnp.float32)
        for c in range(nblk):
            l, o, of = chunk(c)
            mxr = lax.dot_general(mx, of, (((0,), (0,)), ((), ())),
                                  preferred_element_type=jnp.float32)
            ssr = lax.dot_general(ss, of, (((0,), (0,)), ((), ())),
                                  preferred_element_type=jnp.float32)
            p = jnp.exp(l - mxr) / ssr
            prob_ref[c] = p.reshape(1, R)
            pm = jnp.maximum(pm, jnp.max(jnp.where(o, p, NEG), axis=1,
                                         keepdims=True))

        im = jnp.full((GB, 1), BIG, jnp.int32)
        for c in range(nblk):
            _, o, _ = chunk(c)
            p = prob_ref[c].reshape(1, R)
            gidx = c * R + lax.broadcasted_iota(jnp.int32, (1, R), 1)
            cand = jnp.where(o & (p == pm), gidx, BIG)
            im = jnp.minimum(im, jnp.min(cand, axis=1, keepdims=True))

        for c in range(nblk):
            _, o, _ = chunk(c)
            gidx = c * R + lax.broadcasted_iota(jnp.int32, (1, R), 1)
            hit = jnp.where(o & (gidx == im), 1.0, 0.0)
            seed_ref[c] = jnp.max(hit, axis=0, keepdims=True).reshape(1, R)


def _head_call(h, g, batch_ids, W1, b1, ln_g, ln_b, W2, b2):
    N, HID = h.shape
    B = g.shape[0]
    H2 = W1.shape[1]
    R = 2000
    nblk = N // R
    GB = 128
    g_pad = jnp.zeros((GB, HID), jnp.float32).at[:B].set(g)
    bi3 = batch_ids.astype(jnp.int32).reshape(nblk, 1, R)

    full3 = lambda s: pl.BlockSpec(s, lambda i: (0, 0, 0))
    full2 = lambda s: pl.BlockSpec(s, lambda i: (0, 0))
    outs = pl.pallas_call(
        _head_body,
        grid=(nblk,),
        in_specs=[
            pl.BlockSpec((R, HID), lambda i: (i, 0)),
            full3((nblk, 1, R)),
            full2((GB, HID)),
            full2((HID, H2)),
            full2((1, H2)),
            full2((1, H2)),
            full2((1, H2)),
            full2((1, H2)),
            full2((1, 1)),
        ],
        out_specs=[full3((nblk, 1, R)), full3((nblk, 1, R)),
                   full3((nblk, 1, R))],
        out_shape=[jax.ShapeDtypeStruct((nblk, 1, R), jnp.float32)] * 3,
    )(h, bi3, g_pad, W1, b1.reshape(1, H2), ln_g.reshape(1, H2),
      ln_b.reshape(1, H2), W2.reshape(1, H2), b2.reshape(1, 1))
    logits3, prob3, seed3 = outs
    return logits3.reshape(N), prob3.reshape(N), seed3.reshape(N)


_NPAD = 10240
_SLC = _NPAD // 16


def _edge_body(EP, N, edge_hbm, seed_hbm, logits_hbm,
               ew_hbm, em_hbm, ni_hbm,
               src_v, dst_v, out_v, fr_v, nx_v, rc_v, lg_v, ni_v,
               sem, sh_all, sh_comb):
    sid = lax.axis_index("s")
    cid = lax.axis_index("c")
    E = EP * 16
    half = EP // 2
    zeros16 = jnp.zeros((16,), jnp.float32)
    ones16 = jnp.ones((16,), jnp.float32)

    cps = [pltpu.async_copy(edge_hbm.at[pl.ds(sid * EP, EP)], src_v, sem),
           pltpu.async_copy(edge_hbm.at[pl.ds(E + sid * EP, EP)], dst_v, sem),
           pltpu.async_copy(seed_hbm, fr_v.at[pl.ds(0, N)], sem),
           pltpu.async_copy(seed_hbm, rc_v.at[pl.ds(0, N)], sem),
           pltpu.async_copy(logits_hbm, lg_v.at[pl.ds(0, N)], sem)]

    @plsc.parallel_loop(0, _NPAD, 16, unroll=8)
    def _zero(t):
        t = pl.multiple_of(t, 16)
        nx_v[pl.ds(t, 16)] = zeros16
        ni_v[pl.ds(t, 16)] = zeros16

    @plsc.parallel_loop(N, _NPAD, 16)
    def _zero_tail(t):
        t = pl.multiple_of(t, 16)
        fr_v[pl.ds(t, 16)] = zeros16
        rc_v[pl.ds(t, 16)] = zeros16

    for cp in cps:
        cp.wait()

    def combine(part_v):
        pltpu.sync_copy(part_v, sh_all.at[pl.ds(sid * _NPAD, _NPAD)])
        plsc.subcore_barrier()
        rds = [pltpu.async_copy(
            sh_all.at[pl.ds(c * _NPAD + sid * _SLC, _SLC)],
            out_v.at[pl.ds(c * _SLC, _SLC)], sem) for c in range(16)]
        for cp in rds:
            cp.wait()

        @plsc.parallel_loop(0, _SLC, 16, unroll=4)
        def _sum(t):
            t = pl.multiple_of(t, 16)
            acc = out_v[pl.ds(t, 16)]
            for c in range(1, 16):
                acc = acc + out_v[pl.ds(c * _SLC + t, 16)]
            out_v[pl.ds(t, 16)] = acc

    def hop():
        @plsc.parallel_loop(0, EP, 16, unroll=8)
        def _scan(e):
            e = pl.multiple_of(e, 16)
            d = dst_v[pl.ds(e, 16)]
            fm = plsc.load_gather(fr_v, [d])
            s = src_v[pl.ds(e, 16)]
            plsc.store_scatter(nx_v, [s], ones16, mask=fm > 0.0)

        combine(nx_v)
        pltpu.sync_copy(out_v.at[pl.ds(0, _SLC)],
                        sh_comb.at[pl.ds(sid * _SLC, _SLC)])
        plsc.subcore_barrier()
        pltpu.sync_copy(sh_comb, fr_v)

        @plsc.parallel_loop(0, _NPAD, 16, unroll=8)
        def _upd(t):
            t = pl.multiple_of(t, 16)
            rc_v[pl.ds(t, 16)] = rc_v[pl.ds(t, 16)] + fr_v[pl.ds(t, 16)]
            nx_v[pl.ds(t, 16)] = zeros16

    hop()
    hop()

    def fin_scan(base, with_outputs):
        @plsc.parallel_loop(0, half, 16, unroll=4)
        def _fin(e):
            e = pl.multiple_of(e, 16)
            s = src_v[pl.ds(base + e, 16)]
            d = dst_v[pl.ds(base + e, 16)]
            rs = plsc.load_gather(rc_v, [s])
            rd = plsc.load_gather(rc_v, [d])
            m = (rs > 0.0) & (rd > 0.0)
            plsc.store_scatter(ni_v, [s], ones16, mask=m)
            plsc.store_scatter(ni_v, [d], ones16, mask=m)
            if with_outputs:
                ls = plsc.load_gather(lg_v, [s])
                ld = plsc.load_gather(lg_v, [d])
                out_v[pl.ds(e, 16)] = jnp.where(m, ls + ld, 0.0)
                out_v[pl.ds(half + e, 16)] = jnp.where(m, ones16, zeros16)

    base_mine = pl.multiple_of(cid * half, 16)
    base_other = pl.multiple_of((1 - cid) * half, 16)
    fin_scan(base_mine, True)
    pltpu.sync_copy(out_v.at[pl.ds(0, half)],
                    ew_hbm.at[pl.ds(sid * EP + base_mine, half)])
    pltpu.sync_copy(out_v.at[pl.ds(half, half)],
                    em_hbm.at[pl.ds(sid * EP + base_mine, half)])
    fin_scan(base_other, False)

    combine(ni_v)

    @pl.when(cid == 0)
    def _wr_ni():
        pltpu.sync_copy(out_v.at[pl.ds(0, _SLC)],
                        ni_hbm.at[pl.ds(sid * _SLC, _SLC)])


def _edge_call(edge_flat, seed, logits):
    E = edge_flat.shape[0] // 2
    N = seed.shape[0]
    EP = E // 16
    mesh = plsc.VectorSubcoreMesh(core_axis_name="c", subcore_axis_name="s",
                                  num_cores=2, num_subcores=16)
    f32 = jnp.float32
    kern = functools.partial(
        pl.kernel,
        out_type=[jax.ShapeDtypeStruct((E,), f32),
                  jax.ShapeDtypeStruct((E,), f32),
                  jax.ShapeDtypeStruct((_NPAD,), f32)],
        mesh=mesh,
        compiler_params=pltpu.CompilerParams(needs_layout_passes=False),
        scratch_types=[
            pltpu.VMEM((EP,), jnp.int32),
            pltpu.VMEM((EP,), jnp.int32),
            pltpu.VMEM((EP,), f32),
            pltpu.VMEM((_NPAD,), f32),
            pltpu.VMEM((_NPAD,), f32),
            pltpu.VMEM((_NPAD,), f32),
            pltpu.VMEM((_NPAD,), f32),
            pltpu.VMEM((_NPAD,), f32),
            pltpu.SemaphoreType.DMA,
            pltpu.VMEM_SHARED((16 * _NPAD,), f32),
            pltpu.VMEM_SHARED((_NPAD,), f32),
        ],
    )(functools.partial(_edge_body, EP, N))
    return kern(edge_flat, seed, logits)


def kernel(h, g, batch_ids, edge_index, W1, b1, ln_g, ln_b, W2, b2):
    N = h.shape[0]
    logits, node_prob, seed = _head_call(h, g, batch_ids, W1, b1,
                                         ln_g, ln_b, W2, b2)
    edge_flat = edge_index.astype(jnp.int32).reshape(-1)
    ew, emf, nic = _edge_call(edge_flat, seed, logits)
    edge_mask = emf > 0.0
    node_in_mask = nic[:N] > 0.0
    return ew, node_prob, edge_mask, node_in_mask


# speedup vs baseline: 185.9475x; 1.0129x over previous
import functools

import jax
import jax.numpy as jnp
from jax import lax
from jax.experimental import pallas as pl
from jax.experimental.pallas import tpu as pltpu
from jax.experimental.pallas import tpu_sc as plsc

NEG = -1e30
BIG = 2**30


def _head_body(h_ref, bi_ref, g_ref, w1_ref, b1_ref, lng_ref, lnb_ref,
               w2_ref, b2_ref, logits_ref, prob_ref, seed_ref):
    nblk = pl.num_programs(0)
    i = pl.program_id(0)
    R = h_ref.shape[0]
    GB = g_ref.shape[0]

    bi = bi_ref[pl.ds(i, 1)].reshape(1, R)
    one = (bi == lax.broadcasted_iota(jnp.int32, (GB, R), 0))
    onef = one.astype(jnp.float32)
    g_rep = lax.dot_general(onef, g_ref[...], (((0,), (0,)), ((), ())),
                            preferred_element_type=jnp.float32)
    z = h_ref[...] + g_rep
    u = jnp.dot(z, w1_ref[...], preferred_element_type=jnp.float32) + b1_ref[...]
    H2 = u.shape[1]
    mu = jnp.sum(u, axis=-1, keepdims=True) * (1.0 / H2)
    d = u - mu
    var = jnp.sum(d * d, axis=-1, keepdims=True) * (1.0 / H2)
    un = d * lax.rsqrt(var + 1e-5) * lng_ref[...] + lnb_ref[...]
    ur = jnp.maximum(un, 0.0)
    lrow = lax.dot_general(w2_ref[...], ur, (((1,), (1,)), ((), ())),
                           preferred_element_type=jnp.float32) + b2_ref[...]
    logits_ref[pl.ds(i, 1)] = lrow.reshape(1, 1, R)

    @pl.when(i == nblk - 1)
    def _segment_stage():
        def chunk(c):
            l = logits_ref[c].reshape(1, R)
            b = bi_ref[c].reshape(1, R)
            o = (b == lax.broadcasted_iota(jnp.int32, (GB, R), 0))
            return l, o, o.astype(jnp.float32)

        mx = jnp.full((GB, 1), NEG, jnp.float32)
        for c in range(nblk):
            l, o, _ = chunk(c)
            mx = jnp.maximum(mx, jnp.max(jnp.where(o, l, NEG), axis=1,
                                         keepdims=True))

        ss = jnp.zeros((GB, 1), jnp.float32)
        for c in range(nblk):
            l, o, of = chunk(c)
            mxr = lax.dot_general(mx, of, (((0,), (0,)), ((), ())),
                                  preferred_element_type=jnp.float32)
            ex = jnp.exp(l - mxr)
            ss = ss + jnp.sum(jnp.where(o, ex, 0.0), axis=1, keepdims=True)

        pm = jnp.full((GB, 1), NEG, jnp.float32)
        for c in range(nblk):
            l, o, of = chunk(c)
            mxr = lax.dot_general(mx, of, (((0,), (0,)), ((), ())),
                                  preferred_element_type=jnp.float32)
            ssr = lax.dot_general(ss, of, (((0,), (0,)), ((), ())),
                                  preferred_element_type=jnp.float32)
            p = jnp.exp(l - mxr) / ssr
            prob_ref[c] = p.reshape(1, R)
            pm = jnp.maximum(pm, jnp.max(jnp.where(o, p, NEG), axis=1,
                                         keepdims=True))

        im = jnp.full((GB, 1), BIG, jnp.int32)
        for c in range(nblk):
            _, o, _ = chunk(c)
            p = prob_ref[c].reshape(1, R)
            gidx = c * R + lax.broadcasted_iota(jnp.int32, (1, R), 1)
            cand = jnp.where(o & (p == pm), gidx, BIG)
            im = jnp.minimum(im, jnp.min(cand, axis=1, keepdims=True))

        for c in range(nblk):
            _, o, _ = chunk(c)
            gidx = c * R + lax.broadcasted_iota(jnp.int32, (1, R), 1)
            hit = jnp.where(o & (gidx == im), 1.0, 0.0)
            seed_ref[c] = jnp.max(hit, axis=0, keepdims=True).reshape(1, R)


def _head_call(h, g, batch_ids, W1, b1, ln_g, ln_b, W2, b2):
    N, HID = h.shape
    B = g.shape[0]
    H2 = W1.shape[1]
    R = 2000
    nblk = N // R
    GB = 128
    g_pad = jnp.zeros((GB, HID), jnp.float32).at[:B].set(g)
    bi3 = batch_ids.astype(jnp.int32).reshape(nblk, 1, R)

    full3 = lambda s: pl.BlockSpec(s, lambda i: (0, 0, 0))
    full2 = lambda s: pl.BlockSpec(s, lambda i: (0, 0))
    outs = pl.pallas_call(
        _head_body,
        grid=(nblk,),
        in_specs=[
            pl.BlockSpec((R, HID), lambda i: (i, 0)),
            full3((nblk, 1, R)),
            full2((GB, HID)),
            full2((HID, H2)),
            full2((1, H2)),
            full2((1, H2)),
            full2((1, H2)),
            full2((1, H2)),
            full2((1, 1)),
        ],
        out_specs=[full3((nblk, 1, R)), full3((nblk, 1, R)),
                   full3((nblk, 1, R))],
        out_shape=[jax.ShapeDtypeStruct((nblk, 1, R), jnp.float32)] * 3,
    )(h, bi3, g_pad, W1, b1.reshape(1, H2), ln_g.reshape(1, H2),
      ln_b.reshape(1, H2), W2.reshape(1, H2), b2.reshape(1, 1))
    logits3, prob3, seed3 = outs
    return logits3.reshape(N), prob3.reshape(N), seed3.reshape(N)


_NPAD = 10240
_SLC = _NPAD // 16


def _edge_body(EP, N, edge_hbm, seed_hbm, logits_hbm,
               ew_hbm, em_hbm, ni_hbm,
               src_v, dst_v, out_v, fr_v, nx_v, rc_v, lg_v, ni_v,
               sem, sh_all, sh_comb):
    sid = lax.axis_index("s")
    cid = lax.axis_index("c")
    E = EP * 16
    half = EP // 2
    zeros16 = jnp.zeros((16,), jnp.float32)
    ones16 = jnp.ones((16,), jnp.float32)

    cps = [pltpu.async_copy(edge_hbm.at[pl.ds(sid * EP, EP)], src_v, sem),
           pltpu.async_copy(edge_hbm.at[pl.ds(E + sid * EP, EP)], dst_v, sem),
           pltpu.async_copy(seed_hbm, fr_v.at[pl.ds(0, N)], sem),
           pltpu.async_copy(seed_hbm, rc_v.at[pl.ds(0, N)], sem),
           pltpu.async_copy(logits_hbm, lg_v.at[pl.ds(0, N)], sem)]

    @plsc.parallel_loop(0, _NPAD, 16, unroll=8)
    def _zero(t):
        t = pl.multiple_of(t, 16)
        nx_v[pl.ds(t, 16)] = zeros16
        ni_v[pl.ds(t, 16)] = zeros16

    @plsc.parallel_loop(N, _NPAD, 16)
    def _zero_tail(t):
        t = pl.multiple_of(t, 16)
        fr_v[pl.ds(t, 16)] = zeros16
        rc_v[pl.ds(t, 16)] = zeros16

    for cp in cps:
        cp.wait()

    def combine(part_v):
        pltpu.sync_copy(part_v, sh_all.at[pl.ds(sid * _NPAD, _NPAD)])
        plsc.subcore_barrier()
        rds = [pltpu.async_copy(
            sh_all.at[pl.ds(c * _NPAD + sid * _SLC, _SLC)],
            out_v.at[pl.ds(c * _SLC, _SLC)], sem) for c in range(16)]
        for cp in rds:
            cp.wait()

        @plsc.parallel_loop(0, _SLC, 16, unroll=4)
        def _sum(t):
            t = pl.multiple_of(t, 16)
            acc = out_v[pl.ds(t, 16)]
            for c in range(1, 16):
                acc = acc + out_v[pl.ds(c * _SLC + t, 16)]
            out_v[pl.ds(t, 16)] = acc

    def hop():
        @plsc.parallel_loop(0, EP, 16, unroll=8)
        def _scan(e):
            e = pl.multiple_of(e, 16)
            d = dst_v[pl.ds(e, 16)]
            fm = plsc.load_gather(fr_v, [d])
            s = src_v[pl.ds(e, 16)]
            plsc.store_scatter(nx_v, [s], ones16, mask=fm > 0.0)

        combine(nx_v)
        pltpu.sync_copy(out_v.at[pl.ds(0, _SLC)],
                        sh_comb.at[pl.ds(sid * _SLC, _SLC)])
        plsc.subcore_barrier()
        pltpu.sync_copy(sh_comb, fr_v)

        @plsc.parallel_loop(0, _NPAD, 16, unroll=8)
        def _upd(t):
            t = pl.multiple_of(t, 16)
            rc_v[pl.ds(t, 16)] = rc_v[pl.ds(t, 16)] + fr_v[pl.ds(t, 16)]
            nx_v[pl.ds(t, 16)] = zeros16

    hop()
    hop()

    def fin_scan(base, with_outputs):
        @plsc.parallel_loop(0, half, 16, unroll=4)
        def _fin(e):
            e = pl.multiple_of(e, 16)
            s = src_v[pl.ds(base + e, 16)]
            d = dst_v[pl.ds(base + e, 16)]
            rs = plsc.load_gather(rc_v, [s])
            rd = plsc.load_gather(rc_v, [d])
            m = (rs > 0.0) & (rd > 0.0)
            plsc.store_scatter(ni_v, [s], ones16, mask=m)
            plsc.store_scatter(ni_v, [d], ones16, mask=m)
            if with_outputs:
                ls = plsc.load_gather(lg_v, [s])
                ld = plsc.load_gather(lg_v, [d])
                out_v[pl.ds(e, 16)] = jnp.where(m, ls + ld, 0.0)
                out_v[pl.ds(half + e, 16)] = jnp.where(m, ones16, zeros16)

    base_mine = pl.multiple_of(cid * half, 16)
    base_other = pl.multiple_of((1 - cid) * half, 16)
    fin_scan(base_mine, True)
    wrs = [pltpu.async_copy(out_v.at[pl.ds(0, half)],
                            ew_hbm.at[pl.ds(sid * EP + base_mine, half)], sem),
           pltpu.async_copy(out_v.at[pl.ds(half, half)],
                            em_hbm.at[pl.ds(sid * EP + base_mine, half)], sem)]
    fin_scan(base_other, False)
    for cp in wrs:
        cp.wait()

    combine(ni_v)

    @pl.when(cid == 0)
    def _wr_ni():
        pltpu.sync_copy(out_v.at[pl.ds(0, _SLC)],
                        ni_hbm.at[pl.ds(sid * _SLC, _SLC)])


def _edge_call(edge_flat, seed, logits):
    E = edge_flat.shape[0] // 2
    N = seed.shape[0]
    EP = E // 16
    mesh = plsc.VectorSubcoreMesh(core_axis_name="c", subcore_axis_name="s",
                                  num_cores=2, num_subcores=16)
    f32 = jnp.float32
    kern = functools.partial(
        pl.kernel,
        out_type=[jax.ShapeDtypeStruct((E,), f32),
                  jax.ShapeDtypeStruct((E,), f32),
                  jax.ShapeDtypeStruct((_NPAD,), f32)],
        mesh=mesh,
        compiler_params=pltpu.CompilerParams(needs_layout_passes=False),
        scratch_types=[
            pltpu.VMEM((EP,), jnp.int32),
            pltpu.VMEM((EP,), jnp.int32),
            pltpu.VMEM((EP,), f32),
            pltpu.VMEM((_NPAD,), f32),
            pltpu.VMEM((_NPAD,), f32),
            pltpu.VMEM((_NPAD,), f32),
            pltpu.VMEM((_NPAD,), f32),
            pltpu.VMEM((_NPAD,), f32),
            pltpu.SemaphoreType.DMA,
            pltpu.VMEM_SHARED((16 * _NPAD,), f32),
            pltpu.VMEM_SHARED((_NPAD,), f32),
        ],
    )(functools.partial(_edge_body, EP, N))
    return kern(edge_flat, seed, logits)


def kernel(h, g, batch_ids, edge_index, W1, b1, ln_g, ln_b, W2, b2):
    N = h.shape[0]
    logits, node_prob, seed = _head_call(h, g, batch_ids, W1, b1,
                                         ln_g, ln_b, W2, b2)
    edge_flat = edge_index.astype(jnp.int32).reshape(-1)
    ew, emf, nic = _edge_call(edge_flat, seed, logits)
    edge_mask = emf > 0.0
    node_in_mask = nic[:N] > 0.0
    return ew, node_prob, edge_mask, node_in_mask
